# Initial kernel scaffold; baseline (speedup 1.0000x reference)
import jax
import jax.numpy as jnp
from jax.experimental import pallas as pl


def kernel(x, edge_index, W, b):
    raise NotImplementedError("write your pallas kernel here")


# speedup vs baseline: 17.9059x; 17.9059x over previous
import functools

import jax
import jax.numpy as jnp
from jax import lax
from jax.experimental import pallas as pl
from jax.experimental.pallas import tpu as pltpu
from jax.experimental.pallas import tpu_sc as plsc

_N, _E, _D = 10000, 320000, 128
_NP = 10240
_NC, _NS = 2, 16
_NW = _NC * _NS
_EPW = _E // _NW
_K = 80
_CH = _EPW // _K
_RPT = _NP // _NS
_ZR = 128
_NZ = _RPT // _ZR

_mesh = plsc.VectorSubcoreMesh(core_axis_name="c", subcore_axis_name="s")


def _fill_zero(buf, rows, cols):
    ncol = cols // 16

    def body(i, carry):
        r = i // ncol
        c = i % ncol
        buf[r, pl.ds(c * 16, 16)] = jnp.zeros((16,), jnp.float32)
        return carry

    lax.fori_loop(0, rows * ncol, body, 0)


@functools.partial(
    pl.kernel,
    out_type=jax.ShapeDtypeStruct((_NC, _NP, 16), jnp.float32),
    mesh=_mesh,
    scratch_types=[
        pltpu.VMEM((_K,), jnp.int32),
        pltpu.VMEM((_K, 16), jnp.float32),
        pltpu.VMEM((_RPT, 16), jnp.float32),
        pltpu.VMEM_SHARED((_NP, 16), jnp.float32),
    ],
    compiler_params=pltpu.CompilerParams(use_tc_tiling_on_sc=False),
)
def _deg_kernel(dst_hbm, deg_out, idx_v, ones_v, buf_v, deg_sp):
    cid = lax.axis_index("c")
    sid = lax.axis_index("s")
    wid = sid * _NC + cid

    def fill_ones(i, carry):
        ones_v[i, :] = jnp.ones((16,), jnp.float32)
        return carry

    lax.fori_loop(0, _K, fill_ones, 0)
    _fill_zero(buf_v, _RPT, 16)
    pltpu.sync_copy(buf_v, deg_sp.at[pl.ds(sid * _RPT, _RPT)])
    plsc.subcore_barrier()

    def body(i, carry):
        base = wid * _EPW + i * _K
        pltpu.sync_copy(dst_hbm.at[pl.ds(base, _K)], idx_v)
        pltpu.sync_copy(ones_v, deg_sp.at[idx_v], add=True)
        return carry

    lax.fori_loop(0, _CH, body, 0)
    plsc.subcore_barrier()
    pltpu.sync_copy(deg_sp.at[pl.ds(sid * _RPT, _RPT)], buf_v)
    pltpu.sync_copy(buf_v, deg_out.at[cid, pl.ds(sid * _RPT, _RPT)])


@functools.partial(
    pl.kernel,
    out_type=jax.ShapeDtypeStruct((_NC, _NP, _D), jnp.float32),
    mesh=_mesh,
    scratch_types=[
        pltpu.VMEM((_K,), jnp.int32),
        pltpu.VMEM((_K,), jnp.int32),
        pltpu.VMEM((_K, _D), jnp.float32),
        pltpu.VMEM((_ZR, _D), jnp.float32),
        pltpu.VMEM_SHARED((_NP, _D), jnp.float32),
    ],
    compiler_params=pltpu.CompilerParams(use_tc_tiling_on_sc=False),
)
def _msg_kernel(y_hbm, src_hbm, dst_hbm, acc_out, sidx_v, didx_v, msgs_v, buf_v, acc_sp):
    cid = lax.axis_index("c")
    sid = lax.axis_index("s")
    wid = sid * _NC + cid

    _fill_zero(buf_v, _ZR, _D)
    for t in range(_NZ):
        pltpu.sync_copy(buf_v, acc_sp.at[pl.ds(sid * _RPT + t * _ZR, _ZR)])
    plsc.subcore_barrier()

    def body(i, carry):
        base = wid * _EPW + i * _K
        pltpu.sync_copy(src_hbm.at[pl.ds(base, _K)], sidx_v)
        pltpu.sync_copy(dst_hbm.at[pl.ds(base, _K)], didx_v)
        pltpu.sync_copy(y_hbm.at[sidx_v], msgs_v)
        pltpu.sync_copy(msgs_v, acc_sp.at[didx_v], add=True)
        return carry

    lax.fori_loop(0, _CH, body, 0)
    plsc.subcore_barrier()
    for t in range(_NZ):
        sl = pl.ds(sid * _RPT + t * _ZR, _ZR)
        pltpu.sync_copy(acc_sp.at[sl], buf_v)
        pltpu.sync_copy(buf_v, acc_out.at[cid, sl])


def _prep_body(deg_ref, x_ref, w_ref, y_ref, dis_ref):
    deg = deg_ref[0][:_N, 0:1] + deg_ref[1][:_N, 0:1] + 1.0
    dis = lax.rsqrt(deg)
    xw = jnp.dot(x_ref[...], w_ref[...], preferred_element_type=jnp.float32)
    y_ref[...] = xw * dis
    dis_ref[...] = dis


_prep = pl.pallas_call(
    _prep_body,
    out_shape=(
        jax.ShapeDtypeStruct((_N, _D), jnp.float32),
        jax.ShapeDtypeStruct((_N, 1), jnp.float32),
    ),
)


def _out_body(acc_ref, y_ref, dis_ref, b_ref, out_ref):
    out_ref[...] = (acc_ref[0][:_N] + acc_ref[1][:_N] + y_ref[...]) * dis_ref[...] + b_ref[...]


_outk = pl.pallas_call(
    _out_body,
    out_shape=jax.ShapeDtypeStruct((_N, _D), jnp.float32),
)


@jax.jit
def _run(x, edge_index, W, b):
    src = edge_index[0]
    dst = edge_index[1]
    degp = _deg_kernel(dst)
    y, dis = _prep(degp, x, W)
    accp = _msg_kernel(y, src, dst)
    return _outk(accp, y, dis, b.reshape(1, _D))


def kernel(x, edge_index, W, b):
    return _run(x, edge_index, W, b)
